# baseline (speedup 1.0000x reference)
import functools

import jax
import jax.numpy as jnp
from jax import lax
from jax.experimental import pallas as pl
from jax.experimental.pallas import tpu as pltpu
from jax.experimental.pallas import tpu_sc as plsc

VOCAB = 1000000
DIM = 128
BATCH = 4096
SEQ = 200
PAD_ID = 0
UNK_ID = 1

NC = 2
NS = 16
NW = NC * NS
ROWS_PER_W = BATCH // NW
NCH = DIM // 16
NBUF = 4

_IN_BOUNDS = lax.GatherScatterMode.PROMISE_IN_BOUNDS


def _body(ids_hbm, table_hbm, out_hbm, ids_all, rows_big, t01_v,
          g0, g1, g2, g3, p0, p1):
    gsem = (g0, g1, g2, g3)
    psem = (p0, p1)

    wid = lax.axis_index("s") * NC + lax.axis_index("c")
    iota16 = lax.iota(jnp.int32, 16)

    def xlane(x, idx):
        dnums = lax.GatherDimensionNumbers(
            offset_dims=(), collapsed_slice_dims=(0,), start_index_map=(0,))
        return lax.gather(x, idx[:, None], dnums, slice_sizes=(1,),
                          mode=_IN_BOUNDS)

    def hsum_splat(x):
        for sh in (1, 2, 4, 8):
            x = x + xlane(x, iota16 ^ sh)
        return x

    def fire_gather(i, k):
        base = i * SEQ
        dst = k * SEQ
        pltpu.make_async_copy(
            table_hbm.at[ids_all.at[pl.ds(base, 128)]],
            rows_big.at[pl.ds(dst, 128)], gsem[k]).start()
        pltpu.make_async_copy(
            table_hbm.at[ids_all.at[pl.ds(base + 128, 72)]],
            rows_big.at[pl.ds(dst + 128, 72)], gsem[k]).start()

    def wait_gather(k):
        pltpu.make_async_copy(
            table_hbm.at[pl.ds(0, SEQ)],
            rows_big.at[pl.ds(k * SEQ, SEQ)], gsem[k]).wait()

    def fire_pair_scatter(i, pair):
        b = wid * ROWS_PER_W + i
        pltpu.make_async_copy(
            rows_big.at[pl.ds(pair * 2 * SEQ, 2 * SEQ)],
            out_hbm.at[pl.ds((b - 1) * SEQ, 2 * SEQ)], psem[pair]).start()

    def wait_pair_scatter(pair):
        pltpu.make_async_copy(
            rows_big.at[pl.ds(0, 2 * SEQ)],
            out_hbm.at[pl.ds(0, 2 * SEQ)], psem[pair]).wait()

    pltpu.sync_copy(ids_hbm.at[pl.ds(wid * ROWS_PER_W * SEQ, ROWS_PER_W * SEQ)],
                    ids_all)
    pltpu.sync_copy(table_hbm.at[pl.ds(0, 2)], t01_v)

    fire_gather(0, 0)
    fire_gather(1, 1)

    def group_body(g, _):
        for k in range(NBUF):
            i = g * NBUF + k
            base = i * SEQ
            rbase = k * SEQ

            wait_gather(k)

            def sum_body(s, accs):
                out = accs
                for u in range(4):
                    out = tuple(
                        out[c] + rows_big[rbase + s * 4 + u, pl.ds(c * 16, 16)]
                        for c in range(NCH))
                return out
            accs = lax.fori_loop(
                0, SEQ // 4, sum_body,
                tuple(jnp.zeros((16,), jnp.float32) for _ in range(NCH)))

            m01 = jnp.zeros((16,), jnp.int32)
            for j in range(12):
                v = ids_all[pl.ds(base + j * 16, 16)]
                m01 = m01 + jnp.where(v == PAD_ID, 256, 0)
                m01 = m01 + jnp.where(v == UNK_ID, 1, 0)
            vt = ids_all[pl.ds(base + SEQ - 16, 16)]
            tl = iota16 >= 8
            m01 = m01 + jnp.where((vt == PAD_ID) & tl, 256, 0)
            m01 = m01 + jnp.where((vt == UNK_ID) & tl, 1, 0)
            m01v = hsum_splat(m01)
            n0v = m01v >> 8
            n1v = m01v & 255
            countv = SEQ - n0v - n1v
            count = countv[0]
            n0f = n0v.astype(jnp.float32)
            n1f = n1v.astype(jnp.float32)
            countf = countv.astype(jnp.float32)
            scalev = jnp.where(countv > 0, 1.0 / (countf + 1e-8),
                               jnp.zeros((16,), jnp.float32))

            avg = tuple(
                (accs[c]
                 - n0f * t01_v[0, pl.ds(c * 16, 16)]
                 - n1f * t01_v[1, pl.ds(c * 16, 16)]) * scalev
                for c in range(NCH))

            @pl.when(count < SEQ)
            def _():
                def ov_chunk(vv, lane_ok, pos0):
                    ovs = jnp.where(
                        ((vv == PAD_ID) | (vv == UNK_ID)) & lane_ok, 1, 0)
                    novr = hsum_splat(ovs)[0]

                    @pl.when(novr > 0)
                    def _():
                        for p in range(16):
                            @pl.when(ovs[p] > 0)
                            def _():
                                pos = rbase + pos0 + p
                                for c in range(NCH):
                                    rows_big[pos, pl.ds(c * 16, 16)] = avg[c]

                def ov_body(j, _):
                    vv = ids_all[pl.ds(base + j * 16, 16)]
                    ov_chunk(vv, iota16 >= 0, j * 16)
                    return 0
                lax.fori_loop(0, 12, ov_body, 0)
                ov_chunk(vt, tl, SEQ - 16)

            if k % 2 == 1:
                fire_pair_scatter(i, k // 2)

            k2 = (k + 2) % NBUF

            @pl.when(i + 2 < ROWS_PER_W)
            def _():
                if k2 == 2:
                    @pl.when(i >= 4)
                    def _():
                        wait_pair_scatter(1)
                elif k2 == 0:
                    wait_pair_scatter(0)
                fire_gather(i + 2, k2)
        return 0

    lax.fori_loop(0, ROWS_PER_W // NBUF, group_body, 0)

    wait_pair_scatter(0)
    wait_pair_scatter(1)


def kernel(input_ids, table):
    mesh = plsc.VectorSubcoreMesh(core_axis_name="c", subcore_axis_name="s")
    k = functools.partial(
        pl.kernel,
        mesh=mesh,
        out_type=jax.ShapeDtypeStruct((BATCH * SEQ, DIM), jnp.float32),
        scratch_types=(
            [pltpu.VMEM((ROWS_PER_W * SEQ,), jnp.int32),
             pltpu.VMEM((NBUF * SEQ, DIM), jnp.float32),
             pltpu.VMEM((2, DIM), jnp.float32)]
            + [pltpu.SemaphoreType.DMA for _ in range(NBUF + 2)]
        ),
    )(_body)
    out = k(input_ids.reshape(-1), table)
    return out.reshape(BATCH, SEQ, DIM)


# speedup vs baseline: 1.0043x; 1.0043x over previous
import functools

import jax
import jax.numpy as jnp
from jax import lax
from jax.experimental import pallas as pl
from jax.experimental.pallas import tpu as pltpu
from jax.experimental.pallas import tpu_sc as plsc

VOCAB = 1000000
DIM = 128
BATCH = 4096
SEQ = 200
PAD_ID = 0
UNK_ID = 1

NC = 2
NS = 16
NW = NC * NS
ROWS_PER_W = BATCH // NW
NCH = DIM // 16
SEQ_PAD = 208
NBUF = 5

_IN_BOUNDS = lax.GatherScatterMode.PROMISE_IN_BOUNDS


def _body(ids_hbm, table_hbm, out_hbm,
          r0, r1, r2, r3, r4, i0, i1, i2, i3, i4, t01_v,
          g0, g1, g2, g3, g4, s0, s1, s2, s3, s4, d0, d1, d2, d3, d4):
    rows_bufs = (r0, r1, r2, r3, r4)
    ids_bufs = (i0, i1, i2, i3, i4)
    gsem = (g0, g1, g2, g3, g4)
    ssem = (s0, s1, s2, s3, s4)
    isem = (d0, d1, d2, d3, d4)

    wid = lax.axis_index("s") * NC + lax.axis_index("c")
    iota16 = lax.iota(jnp.int32, 16)

    def xlane(x, idx):
        dnums = lax.GatherDimensionNumbers(
            offset_dims=(), collapsed_slice_dims=(0,), start_index_map=(0,))
        return lax.gather(x, idx[:, None], dnums, slice_sizes=(1,),
                          mode=_IN_BOUNDS)

    def hsum_splat(x):
        for sh in (1, 2, 4, 8):
            x = x + xlane(x, iota16 ^ sh)
        return x

    def fire_ids(i, k):
        b = wid * ROWS_PER_W + i
        pltpu.make_async_copy(ids_hbm.at[pl.ds(b * SEQ, SEQ)],
                              ids_bufs[k].at[pl.ds(0, SEQ)], isem[k]).start()

    def wait_ids(k):
        pltpu.make_async_copy(ids_hbm.at[pl.ds(0, SEQ)],
                              ids_bufs[k].at[pl.ds(0, SEQ)], isem[k]).wait()

    def fire_gather(k):
        pltpu.make_async_copy(
            table_hbm.at[ids_bufs[k].at[pl.ds(0, 128)]],
            rows_bufs[k].at[pl.ds(0, 128)], gsem[k]).start()
        pltpu.make_async_copy(
            table_hbm.at[ids_bufs[k].at[pl.ds(128, 72)]],
            rows_bufs[k].at[pl.ds(128, 72)], gsem[k]).start()

    def wait_gather(k):
        pltpu.make_async_copy(
            table_hbm.at[pl.ds(0, SEQ)], rows_bufs[k], gsem[k]).wait()

    def wait_scatter(k):
        pltpu.make_async_copy(
            rows_bufs[k], out_hbm.at[0], ssem[k]).wait()

    pltpu.sync_copy(table_hbm.at[pl.ds(0, 2)], t01_v)

    fire_ids(0, 0)
    fire_ids(1, 1)
    fire_ids(2, 2)
    fire_ids(3, 3)
    wait_ids(0)
    fire_gather(0)
    wait_ids(1)
    fire_gather(1)
    wait_ids(2)
    fire_gather(2)

    def group_body(g, _):
        for k in range(NBUF):
            i = g * NBUF + k
            rows_v = rows_bufs[k]
            ids_v = ids_bufs[k]

            @pl.when(i < ROWS_PER_W)
            def _process():
                wait_gather(k)

                tail = ids_v[pl.ds(192, 16)]
                ids_v[pl.ds(192, 16)] = jnp.where(iota16 < 8, tail, 2)

                def sum_body(s, accs):
                    out = accs
                    for u in range(4):
                        out = tuple(out[c] + rows_v[s * 4 + u, pl.ds(c * 16, 16)]
                                    for c in range(NCH))
                    return out
                accs = lax.fori_loop(
                    0, SEQ // 4, sum_body,
                    tuple(jnp.zeros((16,), jnp.float32) for _ in range(NCH)))

                m01 = jnp.zeros((16,), jnp.int32)
                for j in range(SEQ_PAD // 16):
                    v = ids_v[pl.ds(j * 16, 16)]
                    m01 = m01 + jnp.where(v == PAD_ID, 256, 0)
                    m01 = m01 + jnp.where(v == UNK_ID, 1, 0)
                m01v = hsum_splat(m01)
                n0v = m01v >> 8
                n1v = m01v & 255
                countv = SEQ - n0v - n1v
                count = countv[0]
                n0f = n0v.astype(jnp.float32)
                n1f = n1v.astype(jnp.float32)
                countf = countv.astype(jnp.float32)
                scalev = jnp.where(countv > 0, 1.0 / (countf + 1e-8),
                                   jnp.zeros((16,), jnp.float32))

                avg = tuple(
                    (accs[c]
                     - n0f * t01_v[0, pl.ds(c * 16, 16)]
                     - n1f * t01_v[1, pl.ds(c * 16, 16)]) * scalev
                    for c in range(NCH))

                @pl.when(count < SEQ)
                def _():
                    def ov_body(j, _):
                        v = ids_v[pl.ds(j * 16, 16)]
                        ovs = jnp.where((v == PAD_ID) | (v == UNK_ID), 1, 0)
                        novr = hsum_splat(ovs)[0]

                        @pl.when(novr > 0)
                        def _():
                            for p in range(16):
                                @pl.when(ovs[p] > 0)
                                def _():
                                    pos = j * 16 + p
                                    for c in range(NCH):
                                        rows_v[pos, pl.ds(c * 16, 16)] = avg[c]
                        return 0
                    lax.fori_loop(0, SEQ_PAD // 16, ov_body, 0)

                b = wid * ROWS_PER_W + i
                pltpu.make_async_copy(rows_v, out_hbm.at[b], ssem[k]).start()

                k3 = (k + 3) % NBUF
                k4 = (k + 4) % NBUF

                @pl.when(i + 3 < ROWS_PER_W)
                def _():
                    @pl.when(i >= 2)
                    def _():
                        wait_scatter(k3)
                    wait_ids(k3)
                    fire_gather(k3)

                @pl.when(i + 4 < ROWS_PER_W)
                def _():
                    fire_ids(i + 4, k4)
        return 0

    lax.fori_loop(0, (ROWS_PER_W + NBUF - 1) // NBUF, group_body, 0)

    for k in range(NBUF):
        wait_scatter(k)


def kernel(input_ids, table):
    mesh = plsc.VectorSubcoreMesh(core_axis_name="c", subcore_axis_name="s")
    k = functools.partial(
        pl.kernel,
        mesh=mesh,
        out_type=jax.ShapeDtypeStruct((BATCH, SEQ, DIM), jnp.float32),
        scratch_types=(
            [pltpu.VMEM((SEQ, DIM), jnp.float32) for _ in range(NBUF)]
            + [pltpu.VMEM((SEQ_PAD,), jnp.int32) for _ in range(NBUF)]
            + [pltpu.VMEM((2, DIM), jnp.float32)]
            + [pltpu.SemaphoreType.DMA for _ in range(3 * NBUF)]
        ),
    )(_body)
    return k(input_ids.reshape(-1), table)
